# baseline (speedup 1.0000x reference)
import functools

import jax
import jax.numpy as jnp
from jax import lax
from jax.experimental import pallas as pl
from jax.experimental.pallas import tpu as pltpu
from jax.experimental.pallas import tpu_sc as plsc

B, S, F = 4096, 200, 27
N_CAT, E = 1000, 32
OUT_F = E + (F - 1)
L = 16
NC, NS = 2, 16
NW = NC * NS
SB = B // NW
RS = 8
GPC = RS * SB // L
NCH = S // RS
PAIRS = (NCH - 1) // 2


def _build_sc_call():
    mesh = plsc.VectorSubcoreMesh(core_axis_name="c", subcore_axis_name="s")

    @functools.partial(
        pl.kernel,
        mesh=mesh,
        compiler_params=pltpu.CompilerParams(needs_layout_passes=False),
        out_type=jax.ShapeDtypeStruct((OUT_F, S, B), jnp.float32),
        scratch_types=[
            pltpu.VMEM((E * N_CAT,), jnp.float32),
            pltpu.VMEM((S, SB), jnp.float32),
            pltpu.VMEM((E, RS, SB), jnp.float32),
            pltpu.VMEM((E, RS, SB), jnp.float32),
            pltpu.SemaphoreType.DMA,
            pltpu.SemaphoreType.DMA,
            pltpu.SemaphoreType.DMA,
            pltpu.SemaphoreType.DMA,
            pltpu.SemaphoreType.DMA,
        ],
    )
    def sc_fn(in_hbm, tab_hbm, out_hbm, tab_v, x_v, e0, e1,
              sem_t, sem_x, sem_r, sem_o0, sem_o1):
        wid = lax.axis_index("s") * NC + lax.axis_index("c")
        bw = wid * SB

        pltpu.async_copy(tab_hbm, tab_v, sem_t)
        pltpu.async_copy(in_hbm.at[0, :, pl.ds(bw, SB)], x_v, sem_x)
        pltpu.async_copy(in_hbm.at[pl.ds(1, F - 1), :, pl.ds(bw, SB)],
                         out_hbm.at[pl.ds(E, F - 1), :, pl.ds(bw, SB)], sem_r)
        pltpu.make_async_copy(tab_hbm, tab_v, sem_t).wait()
        pltpu.make_async_copy(
            in_hbm.at[0, :, pl.ds(bw, SB)], x_v, sem_x).wait()

        def compute(s0, e_v):
            def gbody(gi, carry):
                r = gi // (SB // L)
                l0 = (gi % (SB // L)) * L
                x = x_v[s0 + r, pl.ds(l0, L)]
                y = x * 255.0
                t = y + 8388608.0
                rows = plsc.bitcast(t, jnp.int32) & 0x7FFFFF
                for d0 in range(0, E, L):
                    vals = [plsc.load_gather(tab_v, [rows + d * N_CAT])
                            for d in range(d0, d0 + L)]
                    for d in range(d0, d0 + L):
                        e_v[d, r, pl.ds(l0, L)] = vals[d - d0]
                return carry
            lax.fori_loop(0, GPC, gbody, 0)

        def out_slice(s0):
            return out_hbm.at[pl.ds(0, E), pl.ds(s0, RS), pl.ds(bw, SB)]

        def step(p, carry):
            for e_v, sem_o, b in ((e0, sem_o0, 0), (e1, sem_o1, 1)):
                c = 2 * p + b
                s0 = c * RS

                @pl.when(p > 0)
                def _wait_prev(e_v=e_v, s0=s0, sem_o=sem_o):
                    pltpu.make_async_copy(e_v, out_slice(s0), sem_o).wait()

                compute(s0, e_v)
                pltpu.async_copy(e_v, out_slice(s0), sem_o)
            return carry

        lax.fori_loop(0, PAIRS, step, 0)
        s_last = (NCH - 1) * RS
        pltpu.make_async_copy(e0, out_slice(s_last), sem_o0).wait()
        compute(s_last, e0)
        pltpu.async_copy(e0, out_slice(s_last), sem_o0)
        pltpu.make_async_copy(e0, out_slice(s_last), sem_o0).wait()
        pltpu.make_async_copy(e1, out_slice(s_last), sem_o1).wait()
        pltpu.make_async_copy(
            in_hbm.at[pl.ds(1, F - 1), :, pl.ds(bw, SB)],
            out_hbm.at[pl.ds(E, F - 1), :, pl.ds(bw, SB)], sem_r).wait()

    return sc_fn


_sc_call = _build_sc_call()


def kernel(inputs, table):
    in_pm = inputs.transpose(2, 1, 0)
    tab_dm = table.transpose(1, 0).reshape(E * N_CAT)
    out_pm = _sc_call(in_pm, tab_dm)
    return out_pm.transpose(2, 1, 0)
